# baseline (speedup 1.0000x reference)
import functools
import math

import jax
import jax.numpy as jnp
from jax import lax
from jax.experimental import pallas as pl
from jax.experimental.pallas import tpu as pltpu
from jax.experimental.pallas import tpu_sc as plsc

_INFO = plsc.get_sparse_core_info()
_NC = _INFO.num_cores
_NS = _INFO.num_subcores
_L = _INFO.num_lanes
_NW = _NC * _NS

_P = 16
_NBUF = 5
_CH = 128


def _proj_body(t_ref, w_ref, o_ref):
    o_ref[...] = jnp.dot(t_ref[...], w_ref[...],
                         preferred_element_type=jnp.float32)


@functools.lru_cache(maxsize=None)
def _make_sc_pool(B, S, V):
    rows_w = B // _NW
    nidx_w = rows_w * S
    assert nidx_w % _CH == 0
    nch = nidx_w // _CH
    period_idx = (S * _CH) // math.gcd(S, _CH)
    pch = period_idx // _CH
    prow = period_idx // S
    nper = nidx_w // period_idx
    assert pch % _NBUF == 0
    inv_s = 1.0 / S

    mesh = plsc.VectorSubcoreMesh(core_axis_name="c", subcore_axis_name="s")

    scratch = [
        pltpu.VMEM((nch, _CH), jnp.int32),
        pltpu.VMEM((_NBUF, _CH, _P), jnp.float32),
        pltpu.VMEM((rows_w, _P), jnp.float32),
        pltpu.VMEM((_P,), jnp.float32),
    ]
    scratch += [pltpu.SemaphoreType.DMA] * _NBUF

    plan = []
    for k in range(pch):
        lo, hi = k * _CH, (k + 1) * _CH
        segs = []
        j0, j1 = lo // S, (hi - 1) // S
        if j0 == j1:
            segs.append((0, _CH, j0))
        else:
            cut = j1 * S - lo
            segs.append((0, cut, j0))
            segs.append((cut, _CH - cut, j1))
        plan.append(segs)

    @functools.partial(
        pl.kernel,
        out_type=jax.ShapeDtypeStruct((B, _P), jnp.float32),
        mesh=mesh,
        scratch_types=scratch,
        compiler_params=pltpu.CompilerParams(use_tc_tiling_on_sc=False),
    )
    def sc_pool(idx_hbm, proj_hbm, bias_hbm, out_hbm,
                idx_v, bufs, res_v, bias_v, *gsem):
        cid = lax.axis_index("c")
        sid = lax.axis_index("s")
        wid = sid * _NC + cid

        pltpu.sync_copy(idx_hbm.at[pl.ds(wid * nch, nch)], idx_v)
        pltpu.sync_copy(bias_hbm, bias_v)
        bias = bias_v[pl.ds(0, _P)]

        def fire_gather(c, b):
            pltpu.async_copy(proj_hbm.at[idx_v.at[c]], bufs.at[b], gsem[b])

        def wait_gather(b):
            pltpu.make_async_copy(
                proj_hbm.at[idx_v.at[0]], bufs.at[b], gsem[b]).wait()

        for b in range(_NBUF):
            fire_gather(b, b)

        zero = jnp.zeros((_L,), jnp.float32)

        def accum_seg(b, start, length, acc):
            buf = bufs.at[b]
            un = 8

            def body(i, a):
                s0 = start + i * un
                for u in range(un):
                    a = a + buf[s0 + u, pl.ds(0, _P)]
                return a

            return lax.fori_loop(0, length // un, body, acc)

        def period_body(g, carry):
            c0 = g * pch
            acc = zero
            for k in range(pch):
                b = k % _NBUF
                wait_gather(b)
                for (start, length, j) in plan[k]:
                    if (k * _CH + start) % S == 0:
                        acc = zero
                    acc = accum_seg(b, start, length, acc)
                    if (k * _CH + start + length) % S == 0:
                        row = g * prow + j
                        res_v[row, pl.ds(0, _P)] = acc * inv_s + bias

                c5 = c0 + k + _NBUF

                @pl.when(c5 < nch)
                def _():
                    fire_gather(c5, b)

            return carry

        lax.fori_loop(0, nper, period_body, 0)
        pltpu.sync_copy(res_v, out_hbm.at[pl.ds(wid * rows_w, rows_w)])

    return sc_pool


def kernel(x, embed_table, fc_w, fc_b):
    B = x.shape[1]
    S = x.shape[3]
    V, E = embed_table.shape
    C = fc_w.shape[0]

    wp = jnp.zeros((E, _P), jnp.float32).at[:, :C].set(
        fc_w.T.astype(jnp.float32))
    rb = 10000
    assert V % rb == 0
    proj = pl.pallas_call(
        _proj_body,
        grid=(V // rb,),
        in_specs=[
            pl.BlockSpec((rb, E), lambda i: (i, 0)),
            pl.BlockSpec((E, _P), lambda i: (0, 0)),
        ],
        out_specs=pl.BlockSpec((rb, _P), lambda i: (i, 0)),
        out_shape=jax.ShapeDtypeStruct((V, _P), jnp.float32),
    )(embed_table.astype(jnp.float32), wp)

    idx2 = x.reshape(B * S // _CH, _CH).astype(jnp.int32)
    biasp = jnp.zeros((_P,), jnp.float32).at[:C].set(fc_b.astype(jnp.float32))

    out16 = _make_sc_pool(B, S, V)(idx2, proj, biasp)
    return out16[:, :C]


# speedup vs baseline: 1.0026x; 1.0026x over previous
import functools
import math

import jax
import jax.numpy as jnp
from jax import lax
from jax.experimental import pallas as pl
from jax.experimental.pallas import tpu as pltpu
from jax.experimental.pallas import tpu_sc as plsc

_INFO = plsc.get_sparse_core_info()
_NC = _INFO.num_cores
_NS = _INFO.num_subcores
_L = _INFO.num_lanes
_NW = _NC * _NS

_P = 16
_NBUF = 5
_CH = 128


def _proj_body(t_ref, w_ref, o_ref):
    o_ref[...] = jnp.dot(t_ref[...], w_ref[...],
                         preferred_element_type=jnp.float32)


@functools.lru_cache(maxsize=None)
def _make_sc_pool(B, S, V):
    rows_w = B // _NW
    nidx_w = rows_w * S
    assert nidx_w % _CH == 0
    nch = nidx_w // _CH
    period_idx = (S * _CH) // math.gcd(S, _CH)
    pch = period_idx // _CH
    prow = period_idx // S
    nper = nidx_w // period_idx
    assert pch % _NBUF == 0
    inv_s = 1.0 / S

    mesh = plsc.VectorSubcoreMesh(core_axis_name="c", subcore_axis_name="s")

    scratch = [
        pltpu.VMEM((nch, _CH), jnp.int32),
        pltpu.VMEM((_NBUF, _CH, _P), jnp.float32),
        pltpu.VMEM((rows_w, _P), jnp.float32),
        pltpu.VMEM((_P,), jnp.float32),
    ]
    scratch += [pltpu.SemaphoreType.DMA] * _NBUF

    plan = []
    for k in range(pch):
        lo, hi = k * _CH, (k + 1) * _CH
        segs = []
        j0, j1 = lo // S, (hi - 1) // S
        if j0 == j1:
            segs.append((0, _CH, j0))
        else:
            cut = j1 * S - lo
            segs.append((0, cut, j0))
            segs.append((cut, _CH - cut, j1))
        plan.append(segs)

    @functools.partial(
        pl.kernel,
        out_type=jax.ShapeDtypeStruct((B, _P), jnp.float32),
        mesh=mesh,
        scratch_types=scratch,
        compiler_params=pltpu.CompilerParams(use_tc_tiling_on_sc=False),
    )
    def sc_pool(idx_hbm, proj_hbm, bias_hbm, out_hbm,
                idx_v, bufs, res_v, bias_v, *gsem):
        cid = lax.axis_index("c")
        sid = lax.axis_index("s")
        wid = sid * _NC + cid

        pltpu.sync_copy(idx_hbm.at[pl.ds(wid * nch, nch)], idx_v)
        pltpu.sync_copy(bias_hbm, bias_v)
        bias = bias_v[pl.ds(0, _P)]

        def fire_gather(c, b):
            pltpu.async_copy(proj_hbm.at[idx_v.at[c]], bufs.at[b], gsem[b])

        def wait_gather(b):
            pltpu.make_async_copy(
                proj_hbm.at[idx_v.at[0]], bufs.at[b], gsem[b]).wait()

        for b in range(_NBUF):
            fire_gather(b, b)

        zero = jnp.zeros((_L,), jnp.float32)

        def accum_seg(b, start, length, acc):
            buf = bufs.at[b]
            un = 8

            def body(i, a):
                s0 = start + i * un
                for u in range(un):
                    a = a + buf[s0 + u, pl.ds(0, _P)]
                return a

            return lax.fori_loop(0, length // un, body, acc)

        def period_body(g, carry):
            c0 = g * pch
            acc = zero
            for k in range(pch):
                b = k % _NBUF
                wait_gather(b)
                for (start, length, j) in plan[k]:
                    if (k * _CH + start) % S == 0:
                        acc = zero
                    acc = accum_seg(b, start, length, acc)
                    if (k * _CH + start + length) % S == 0:
                        row = g * prow + j
                        res_v[row, pl.ds(0, _P)] = acc * inv_s + bias

                c5 = c0 + k + _NBUF

                @pl.when(c5 < nch)
                def _():
                    fire_gather(c5, b)

            return carry

        lax.fori_loop(0, nper, period_body, 0)
        pltpu.sync_copy(res_v, out_hbm.at[pl.ds(wid * rows_w, rows_w)])

    return sc_pool


def kernel(x, embed_table, fc_w, fc_b):
    B = x.shape[1]
    S = x.shape[3]
    V, E = embed_table.shape
    C = fc_w.shape[0]

    wp = jnp.zeros((E, _P), jnp.float32).at[:, :C].set(
        fc_w.T.astype(jnp.float32))
    rb = 10000
    assert V % rb == 0
    proj = pl.pallas_call(
        _proj_body,
        grid=(V // rb,),
        in_specs=[
            pl.BlockSpec((rb, E), lambda i: (i, 0)),
            pl.BlockSpec((E, _P), lambda i: (0, 0)),
        ],
        out_specs=pl.BlockSpec((rb, _P), lambda i: (i, 0)),
        out_shape=jax.ShapeDtypeStruct((V, _P), jnp.float32),
    )(embed_table.astype(jnp.float32), wp)

    idx2 = lax.optimization_barrier(
        x.reshape(B * S // _CH, _CH).astype(jnp.int32))
    biasp = jnp.zeros((_P,), jnp.float32).at[:C].set(fc_b.astype(jnp.float32))

    out16 = _make_sc_pool(B, S, V)(idx2, proj, biasp)
    return out16[:, :C]


# speedup vs baseline: 1.4506x; 1.4469x over previous
import functools

import jax
import jax.numpy as jnp
from jax import lax
from jax.experimental import pallas as pl
from jax.experimental.pallas import tpu as pltpu
from jax.experimental.pallas import tpu_sc as plsc

_INFO = plsc.get_sparse_core_info()
_NC = _INFO.num_cores
_NS = _INFO.num_subcores
_L = _INFO.num_lanes
_NW = _NC * _NS

_NBUF = 4


@functools.lru_cache(maxsize=None)
def _make_sc_pool(B, S, V, E):
    rows_w = B // _NW
    ech = E // _L
    ca = 104
    cb = S - ca
    assert ca % 8 == 0 and cb % 8 == 0 and S % 8 == 0
    inv_s = 1.0 / S

    mesh = plsc.VectorSubcoreMesh(core_axis_name="c", subcore_axis_name="s")

    scratch = [
        pltpu.VMEM((rows_w, S), jnp.int32),
        pltpu.VMEM((_NBUF, ca, E), jnp.float32),
        pltpu.VMEM((rows_w, E), jnp.float32),
    ]
    scratch += [pltpu.SemaphoreType.DMA] * _NBUF

    @functools.partial(
        pl.kernel,
        out_type=jax.ShapeDtypeStruct((B, E), jnp.float32),
        mesh=mesh,
        scratch_types=scratch,
        compiler_params=pltpu.CompilerParams(use_tc_tiling_on_sc=False),
    )
    def sc_pool(x_hbm, table_hbm, out_hbm, idx_v, bufs, res_v, *gsem):
        cid = lax.axis_index("c")
        sid = lax.axis_index("s")
        wid = sid * _NC + cid

        pltpu.sync_copy(
            x_hbm.at[0, pl.ds(wid * rows_w, rows_w), 0, :], idx_v)

        def fire_row(r, b0):
            pltpu.async_copy(
                table_hbm.at[idx_v.at[r, pl.ds(0, ca)]],
                bufs.at[b0, pl.ds(0, ca)], gsem[b0])
            pltpu.async_copy(
                table_hbm.at[idx_v.at[r, pl.ds(ca, cb)]],
                bufs.at[b0 + 1, pl.ds(0, cb)], gsem[b0 + 1])

        def wait_row(b0):
            pltpu.make_async_copy(
                table_hbm.at[idx_v.at[0, pl.ds(0, ca)]],
                bufs.at[b0, pl.ds(0, ca)], gsem[b0]).wait()
            pltpu.make_async_copy(
                table_hbm.at[idx_v.at[0, pl.ds(ca, cb)]],
                bufs.at[b0 + 1, pl.ds(0, cb)], gsem[b0 + 1]).wait()

        zero = jnp.zeros((_L,), jnp.float32)

        def accum(b0, accs):
            un = 8

            def mk_body(b, base):
                def body(i, a):
                    s0 = base + i * un
                    a = list(a)
                    for u in range(un):
                        for d in range(ech):
                            a[d] = a[d] + bufs[b, s0 + u, pl.ds(d * _L, _L)]
                    return tuple(a)

                return body

            accs = lax.fori_loop(0, ca // un, mk_body(b0, 0), accs)
            accs = lax.fori_loop(0, cb // un, mk_body(b0 + 1, 0), accs)
            return accs

        fire_row(0, 0)
        fire_row(1, 2)

        def row_body(r, carry):
            b0 = 0
            wait_row(b0)
            accs = accum(b0, (zero,) * ech)

            @pl.when(r + 2 < rows_w)
            def _():
                fire_row(r + 2, b0)

            for d in range(ech):
                res_v[r, pl.ds(d * _L, _L)] = accs[d]

            r1 = r + 1
            b1 = 2
            wait_row(b1)
            accs = accum(b1, (zero,) * ech)

            @pl.when(r1 + 2 < rows_w)
            def _():
                fire_row(r1 + 2, b1)

            for d in range(ech):
                res_v[r1, pl.ds(d * _L, _L)] = accs[d]

            return carry

        def pair_body(g, carry):
            return row_body(g * 2, carry)

        lax.fori_loop(0, rows_w // 2, pair_body, 0)
        pltpu.sync_copy(res_v, out_hbm.at[pl.ds(wid * rows_w, rows_w)])

    return sc_pool


def _head_body(s_ref, w_ref, b_ref, o_ref, *, inv_s):
    acc = jnp.dot(s_ref[...], w_ref[...], preferred_element_type=jnp.float32)
    o_ref[...] = acc * inv_s + b_ref[...]


def kernel(x, embed_table, fc_w, fc_b):
    B = x.shape[1]
    S = x.shape[3]
    V, E = embed_table.shape
    C = fc_w.shape[0]

    xi = x.astype(jnp.int32)
    sums = _make_sc_pool(B, S, V, E)(xi, embed_table.astype(jnp.float32))

    head = pl.pallas_call(
        functools.partial(_head_body, inv_s=1.0 / S),
        out_shape=jax.ShapeDtypeStruct((B, C), jnp.float32),
    )
    w_t = jnp.transpose(fc_w).astype(jnp.float32)
    return head(sums, w_t, fc_b.reshape(1, C).astype(jnp.float32))


# speedup vs baseline: 1.4520x; 1.0010x over previous
import functools

import jax
import jax.numpy as jnp
from jax import lax
from jax.experimental import pallas as pl
from jax.experimental.pallas import tpu as pltpu
from jax.experimental.pallas import tpu_sc as plsc

_INFO = plsc.get_sparse_core_info()
_NC = _INFO.num_cores
_NS = _INFO.num_subcores
_L = _INFO.num_lanes
_NW = _NC * _NS

_NBUF = 4


@functools.lru_cache(maxsize=None)
def _make_sc_pool(B, S, V, E):
    rows_w = B // _NW
    ech = E // _L
    ca = 104
    cb = S - ca
    assert ca % 8 == 0 and cb % 8 == 0 and S % 8 == 0
    inv_s = 1.0 / S

    mesh = plsc.VectorSubcoreMesh(core_axis_name="c", subcore_axis_name="s")

    scratch = [
        pltpu.VMEM((rows_w, S), jnp.int32),
        pltpu.VMEM((_NBUF, ca, E), jnp.float32),
        pltpu.VMEM((rows_w, E), jnp.float32),
    ]
    scratch += [pltpu.SemaphoreType.DMA] * _NBUF

    @functools.partial(
        pl.kernel,
        out_type=jax.ShapeDtypeStruct((B, E), jnp.float32),
        mesh=mesh,
        scratch_types=scratch,
        compiler_params=pltpu.CompilerParams(use_tc_tiling_on_sc=False),
    )
    def sc_pool(x_hbm, table_hbm, out_hbm, idx_v, bufs, res_v, *gsem):
        cid = lax.axis_index("c")
        sid = lax.axis_index("s")
        wid = sid * _NC + cid

        pltpu.sync_copy(x_hbm.at[pl.ds(wid * rows_w, rows_w), :], idx_v)

        def fire_row(r, b0):
            pltpu.async_copy(
                table_hbm.at[idx_v.at[r, pl.ds(0, ca)]],
                bufs.at[b0, pl.ds(0, ca)], gsem[b0])
            pltpu.async_copy(
                table_hbm.at[idx_v.at[r, pl.ds(ca, cb)]],
                bufs.at[b0 + 1, pl.ds(0, cb)], gsem[b0 + 1])

        def wait_row(b0):
            pltpu.make_async_copy(
                table_hbm.at[idx_v.at[0, pl.ds(0, ca)]],
                bufs.at[b0, pl.ds(0, ca)], gsem[b0]).wait()
            pltpu.make_async_copy(
                table_hbm.at[idx_v.at[0, pl.ds(ca, cb)]],
                bufs.at[b0 + 1, pl.ds(0, cb)], gsem[b0 + 1]).wait()

        zero = jnp.zeros((_L,), jnp.float32)

        def accum(b0, accs):
            un = 8

            def mk_body(b, base):
                def body(i, a):
                    s0 = base + i * un
                    a = list(a)
                    for u in range(un):
                        for d in range(ech):
                            a[d] = a[d] + bufs[b, s0 + u, pl.ds(d * _L, _L)]
                    return tuple(a)

                return body

            accs = lax.fori_loop(0, ca // un, mk_body(b0, 0), accs)
            accs = lax.fori_loop(0, cb // un, mk_body(b0 + 1, 0), accs)
            return accs

        fire_row(0, 0)
        fire_row(1, 2)

        def row_body(r, carry):
            b0 = 0
            wait_row(b0)
            accs = accum(b0, (zero,) * ech)

            @pl.when(r + 2 < rows_w)
            def _():
                fire_row(r + 2, b0)

            for d in range(ech):
                res_v[r, pl.ds(d * _L, _L)] = accs[d]

            r1 = r + 1
            b1 = 2
            wait_row(b1)
            accs = accum(b1, (zero,) * ech)

            @pl.when(r1 + 2 < rows_w)
            def _():
                fire_row(r1 + 2, b1)

            for d in range(ech):
                res_v[r1, pl.ds(d * _L, _L)] = accs[d]

            return carry

        def pair_body(g, carry):
            return row_body(g * 2, carry)

        lax.fori_loop(0, rows_w // 2, pair_body, 0)
        pltpu.sync_copy(res_v, out_hbm.at[pl.ds(wid * rows_w, rows_w)])

    return sc_pool


def _head_body(s_ref, w_ref, b_ref, o_ref, *, inv_s):
    acc = jnp.dot(s_ref[...], w_ref[...], preferred_element_type=jnp.float32)
    o_ref[...] = acc * inv_s + b_ref[...]


def kernel(x, embed_table, fc_w, fc_b):
    B = x.shape[1]
    S = x.shape[3]
    V, E = embed_table.shape
    C = fc_w.shape[0]

    xi = x.reshape(B, S).astype(jnp.int32)
    sums = _make_sc_pool(B, S, V, E)(xi, embed_table.astype(jnp.float32))

    head = pl.pallas_call(
        functools.partial(_head_body, inv_s=1.0 / S),
        out_shape=jax.ShapeDtypeStruct((B, C), jnp.float32),
    )
    w_t = jnp.transpose(fc_w).astype(jnp.float32)
    return head(sums, w_t, fc_b.reshape(1, C).astype(jnp.float32))
